# Initial kernel scaffold; baseline (speedup 1.0000x reference)
import functools
import jax
import jax.numpy as jnp
from jax import lax
from jax.experimental import pallas as pl
from jax.experimental.pallas import tpu as pltpu
from jax.experimental.pallas import tpu_sc as plsc

N = 10000
NP = 10240
D = 128
H = 64
CH = 128
K = 157
E = 320000
EP = 16 * K * CH
ROWS_PER_TILE = NP // 16

_mesh = plsc.VectorSubcoreMesh(core_axis_name="c", subcore_axis_name="s")


@functools.partial(
    pl.kernel,
    out_type=jax.ShapeDtypeStruct((NP, 16), jnp.float32),
    mesh=_mesh,
    scratch_types=[
        pltpu.VMEM((K, CH), jnp.int32),
        pltpu.VMEM((CH, 16), jnp.float32),
        pltpu.VMEM((ROWS_PER_TILE, 16), jnp.float32),
        pltpu.VMEM_SHARED((NP, 16), jnp.float32),
        pltpu.SemaphoreType.DMA,
    ],
)
def _deg_kernel(cols_hbm, hist_hbm, cidx, ones_v, zbuf, hist_s, sem):
    c = lax.axis_index("c")
    s = lax.axis_index("s")

    @pl.when(c == 0)
    def _():
        ones16 = jnp.ones((16,), jnp.float32)
        zero16 = jnp.zeros((16,), jnp.float32)

        @pl.loop(0, CH)
        def _(i):
            ones_v[i, :] = ones16

        @pl.loop(0, ROWS_PER_TILE)
        def _(i):
            zbuf[i, :] = zero16

        pltpu.sync_copy(cols_hbm.at[s], cidx)
        pltpu.sync_copy(zbuf, hist_s.at[pl.ds(s * ROWS_PER_TILE, ROWS_PER_TILE)])
        plsc.subcore_barrier()

        @pl.loop(0, K)
        def _(j):
            pltpu.sync_copy(ones_v, hist_s.at[cidx.at[j]], add=True)

        plsc.subcore_barrier()
        pltpu.sync_copy(
            hist_s.at[pl.ds(s * ROWS_PER_TILE, ROWS_PER_TILE)],
            hist_hbm.at[pl.ds(s * ROWS_PER_TILE, ROWS_PER_TILE)],
        )


@functools.partial(
    pl.kernel,
    out_type=jax.ShapeDtypeStruct((2 * NP, H), jnp.float32),
    mesh=_mesh,
    scratch_types=[
        pltpu.VMEM((K, CH), jnp.int32),
        pltpu.VMEM((K, CH), jnp.int32),
        pltpu.VMEM((CH, H), jnp.float32),
        pltpu.VMEM_SHARED((NP, H), jnp.float32),
        pltpu.SemaphoreType.DMA,
    ],
)
def _spmv_kernel(y_hbm, rows_hbm, cols_hbm, out_hbm, ridx, cidx, gbuf, z_s, sem):
    c = lax.axis_index("c")
    s = lax.axis_index("s")
    w = c * 16 + s

    pltpu.sync_copy(rows_hbm.at[w], ridx)
    pltpu.sync_copy(cols_hbm.at[s], cidx)

    zero16 = jnp.zeros((16,), jnp.float32)

    @pl.loop(0, CH)
    def _(i):
        gbuf[i, pl.ds(0, 16)] = zero16
        gbuf[i, pl.ds(16, 16)] = zero16
        gbuf[i, pl.ds(32, 16)] = zero16
        gbuf[i, pl.ds(48, 16)] = zero16

    @pl.loop(0, ROWS_PER_TILE // CH)
    def _(k):
        pltpu.sync_copy(gbuf, z_s.at[pl.ds(s * ROWS_PER_TILE + k * CH, CH)])

    plsc.subcore_barrier()

    @pl.loop(0, K)
    def _(j):
        pltpu.async_copy(y_hbm.at[ridx.at[j]], gbuf, sem).wait()
        pltpu.sync_copy(gbuf, z_s.at[cidx.at[j]], add=True)

    plsc.subcore_barrier()
    pltpu.sync_copy(
        z_s.at[pl.ds(s * ROWS_PER_TILE, ROWS_PER_TILE)],
        out_hbm.at[pl.ds(c * NP + s * ROWS_PER_TILE, ROWS_PER_TILE)],
    )


def _prep_body(hist_ref, x_ref, dise_ref, y0_ref):
    deg = hist_ref[:, 0:1] + 1.0
    dis = lax.rsqrt(jnp.maximum(deg, 1.0))
    dise_ref[pl.ds(0, NP), :] = dis
    dise_ref[pl.ds(NP, NP), :] = dis
    y0_ref[pl.ds(0, NP), :] = x_ref[:, 0:H] * dis
    y0_ref[pl.ds(NP, NP), :] = x_ref[:, H:D] * dis


_prep_call = pl.pallas_call(
    _prep_body,
    out_shape=(
        jax.ShapeDtypeStruct((2 * NP, 1), jnp.float32),
        jax.ShapeDtypeStruct((2 * NP, H), jnp.float32),
    ),
)


def _scale_body(s_ref, y_ref, dise_ref, t_ref, ynew_ref):
    tmp = (s_ref[...] + y_ref[...]) * dise_ref[...]
    ynew_ref[...] = tmp * dise_ref[...]
    t_ref[:, 0:H] = tmp[pl.ds(0, NP), :]
    t_ref[:, H:D] = tmp[pl.ds(NP, NP), :]


_scale_call = pl.pallas_call(
    _scale_body,
    out_shape=(
        jax.ShapeDtypeStruct((NP, D), jnp.float32),
        jax.ShapeDtypeStruct((2 * NP, H), jnp.float32),
    ),
)


def _final_body(x_ref, t1_ref, t2_ref, t3_ref, t4_ref, w0t_ref, wts_ref,
                bsum_ref, fw_ref, wct_ref, bc_ref, out_ref):
    acc = jnp.dot(x_ref[...], w0t_ref[...], preferred_element_type=jnp.float32)
    for i, t_ref in enumerate((t1_ref, t2_ref, t3_ref, t4_ref)):
        acc = acc + jnp.dot(t_ref[...], wts_ref[i],
                            preferred_element_type=jnp.float32)
    acc = acc + bsum_ref[...]
    h = jnp.maximum(acc, 0.0)
    ew = jnp.exp(fw_ref[...])
    w0 = ew[0, 0] / (ew[0, 0] + ew[0, 1])
    out_ref[...] = jnp.dot(h * w0, wct_ref[...],
                           preferred_element_type=jnp.float32) + bc_ref[...]


_final_call = pl.pallas_call(
    _final_body,
    out_shape=jax.ShapeDtypeStruct((NP, 64), jnp.float32),
)


@jax.jit
def kernel(x, edge_index, W0, b0, W1, b1, W2, b2, W3, b3, W4, b4,
           fusion_weight, Wc, bc):
    rows = edge_index[0]
    cols = edge_index[1]
    pad = EP - E
    rows_p = jnp.concatenate([rows, jnp.zeros((pad,), jnp.int32)])
    cols_p = jnp.concatenate([cols, jnp.full((pad,), N, jnp.int32)])
    rows2 = jnp.stack([rows_p, rows_p + NP]).reshape(2 * 16, K, CH)
    cols3 = cols_p.reshape(16, K, CH)
    x_pad = jnp.pad(x, ((0, NP - N), (0, 0)))

    w0t = W0.T
    wts = jnp.stack([W1.T, W2.T, W3.T, W4.T])
    bsum = (b0 + b1 + b2 + b3 + b4).reshape(1, D)
    fw = fusion_weight.reshape(1, 2)
    wct = Wc.T
    bc2 = bc.reshape(1, 64)

    hist = _deg_kernel(cols3)
    dise, y = _prep_call(hist, x_pad)

    ts = []
    for _ in range(4):
        s_out = _spmv_kernel(y, rows2, cols3)
        t_i, y = _scale_call(s_out, y, dise)
        ts.append(t_i)

    logits = _final_call(x_pad, ts[0], ts[1], ts[2], ts[3],
                         w0t, wts, bsum, fw, wct, bc2)
    return logits[:N]


# speedup vs baseline: 7.8139x; 7.8139x over previous
import functools
import jax
import jax.numpy as jnp
from jax import lax
from jax.experimental import pallas as pl
from jax.experimental.pallas import tpu as pltpu
from jax.experimental.pallas import tpu_sc as plsc

N = 10000
NP = 10240
D = 128
CH = 128
K = 79
E = 320000
EP = 32 * K * CH
ROWS_PER_TILE = NP // 16

_mesh = plsc.VectorSubcoreMesh(core_axis_name="c", subcore_axis_name="s")


@functools.partial(
    pl.kernel,
    out_type=jax.ShapeDtypeStruct((2 * NP, D), jnp.float32),
    mesh=_mesh,
    scratch_types=[
        pltpu.VMEM((K, CH), jnp.int32),
        pltpu.VMEM((K, CH), jnp.int32),
        pltpu.VMEM((CH, D), jnp.float32),
        pltpu.VMEM_SHARED((NP, D), jnp.float32),
        pltpu.SemaphoreType.DMA,
    ],
)
def _spmv_kernel(y_hbm, rows_hbm, cols_hbm, out_hbm, ridx, cidx, gbuf, z_s, sem):
    c = lax.axis_index("c")
    s = lax.axis_index("s")
    w = c * 16 + s

    pltpu.sync_copy(rows_hbm.at[w], ridx)
    pltpu.sync_copy(cols_hbm.at[w], cidx)

    zero16 = jnp.zeros((16,), jnp.float32)

    @pl.loop(0, CH)
    def _(i):
        for q in range(8):
            gbuf[i, pl.ds(16 * q, 16)] = zero16

    @pl.loop(0, ROWS_PER_TILE // CH)
    def _(k):
        pltpu.sync_copy(gbuf, z_s.at[pl.ds(s * ROWS_PER_TILE + k * CH, CH)])

    plsc.subcore_barrier()

    @pl.loop(0, K)
    def _(j):
        pltpu.async_copy(y_hbm.at[ridx.at[j]], gbuf, sem).wait()
        pltpu.sync_copy(gbuf, z_s.at[cidx.at[j]], add=True)

    plsc.subcore_barrier()
    pltpu.sync_copy(
        z_s.at[pl.ds(s * ROWS_PER_TILE, ROWS_PER_TILE)],
        out_hbm.at[pl.ds(c * NP + s * ROWS_PER_TILE, ROWS_PER_TILE)],
    )


def _prep_body(sdeg_ref, x_ref, dise_ref, y0_ref):
    deg = sdeg_ref[0:NP, 0:1] + sdeg_ref[NP:2 * NP, 0:1] + 1.0
    dis = lax.rsqrt(jnp.maximum(deg, 1.0))
    dise_ref[...] = dis
    y0_ref[...] = x_ref[...] * dis


_prep_call = pl.pallas_call(
    _prep_body,
    out_shape=(
        jax.ShapeDtypeStruct((NP, 1), jnp.float32),
        jax.ShapeDtypeStruct((NP, D), jnp.float32),
    ),
)


def _scale_body(s_ref, y_ref, dise_ref, t_ref, ynew_ref):
    dis = dise_ref[...]
    tmp = (s_ref[0:NP, :] + s_ref[NP:2 * NP, :] + y_ref[...]) * dis
    t_ref[...] = tmp
    ynew_ref[...] = tmp * dis


_scale_call = pl.pallas_call(
    _scale_body,
    out_shape=(
        jax.ShapeDtypeStruct((NP, D), jnp.float32),
        jax.ShapeDtypeStruct((NP, D), jnp.float32),
    ),
)


def _final_body(x_ref, t1_ref, t2_ref, t3_ref, t4_ref, w0t_ref, wts_ref,
                bsum_ref, fw_ref, wct_ref, bc_ref, out_ref):
    acc = jnp.dot(x_ref[...], w0t_ref[...], preferred_element_type=jnp.float32)
    for i, t_ref in enumerate((t1_ref, t2_ref, t3_ref, t4_ref)):
        acc = acc + jnp.dot(t_ref[...], wts_ref[i],
                            preferred_element_type=jnp.float32)
    acc = acc + bsum_ref[...]
    h = jnp.maximum(acc, 0.0)
    ew = jnp.exp(fw_ref[...])
    w0 = ew[0, 0] / (ew[0, 0] + ew[0, 1])
    out_ref[...] = jnp.dot(h * w0, wct_ref[...],
                           preferred_element_type=jnp.float32) + bc_ref[...]


_final_call = pl.pallas_call(
    _final_body,
    out_shape=jax.ShapeDtypeStruct((NP, 64), jnp.float32),
)


@jax.jit
def kernel(x, edge_index, W0, b0, W1, b1, W2, b2, W3, b3, W4, b4,
           fusion_weight, Wc, bc):
    pad = EP - E
    rows_p = jnp.concatenate([edge_index[0], jnp.zeros((pad,), jnp.int32)])
    cols_p = jnp.concatenate([edge_index[1], jnp.full((pad,), N, jnp.int32)])
    rows3 = rows_p.reshape(32, K, CH)
    cols3 = cols_p.reshape(32, K, CH)
    x_pad = jnp.pad(x, ((0, NP - N), (0, 0)))
    ones_feat = jnp.ones((NP, D), jnp.float32)

    w0t = W0.T
    wts = jnp.stack([W1.T, W2.T, W3.T, W4.T])
    bsum = (b0 + b1 + b2 + b3 + b4).reshape(1, D)
    fw = fusion_weight.reshape(1, 2)
    wct = Wc.T
    bc2 = bc.reshape(1, 64)

    sdeg = _spmv_kernel(ones_feat, rows3, cols3)
    dise, y = _prep_call(sdeg, x_pad)

    ts = []
    for _ in range(4):
        s_out = _spmv_kernel(y, rows3, cols3)
        t_i, y = _scale_call(s_out, y, dise)
        ts.append(t_i)

    logits = _final_call(x_pad, ts[0], ts[1], ts[2], ts[3],
                         w0t, wts, bsum, fw, wct, bc2)
    return logits[:N]


# speedup vs baseline: 11.5861x; 1.4828x over previous
import functools
import jax
import jax.numpy as jnp
from jax import lax
from jax.experimental import pallas as pl
from jax.experimental.pallas import tpu as pltpu
from jax.experimental.pallas import tpu_sc as plsc

N = 10000
NP = 10240
D = 128
CH = 112
K = 90
NB = 6
NBLK = K // NB
E = 320000
EP = 32 * K * CH
ROWS_PER_TILE = NP // 16

_mesh = plsc.VectorSubcoreMesh(core_axis_name="c", subcore_axis_name="s")


@functools.partial(
    pl.kernel,
    out_type=jax.ShapeDtypeStruct((2 * NP, D), jnp.float32),
    mesh=_mesh,
    scratch_types=[
        pltpu.VMEM((2, NB, CH), jnp.int32),
        pltpu.VMEM((K, CH), jnp.int32),
        pltpu.VMEM((CH, D), jnp.float32),
        pltpu.VMEM((CH, D), jnp.float32),
        pltpu.VMEM_SHARED((NP, D), jnp.float32),
        pltpu.SemaphoreType.DMA,
        pltpu.SemaphoreType.DMA,
        pltpu.SemaphoreType.DMA,
    ],
)
def _spmv_kernel(y_hbm, rows_hbm, cols_hbm, out_hbm, rblk, cidx, gbufa, gbufb,
                 z_s, sema, semb, semr):
    c = lax.axis_index("c")
    s = lax.axis_index("s")
    w = c * 16 + s

    pltpu.sync_copy(cols_hbm.at[w], cidx)

    zero16 = jnp.zeros((16,), jnp.float32)

    @pl.loop(0, CH)
    def _(i):
        for q in range(8):
            gbufa[i, pl.ds(16 * q, 16)] = zero16

    @pl.loop(0, ROWS_PER_TILE // 80)
    def _(k):
        pltpu.sync_copy(gbufa.at[pl.ds(0, 80)],
                        z_s.at[pl.ds(s * ROWS_PER_TILE + k * 80, 80)])

    plsc.subcore_barrier()

    gbufs = (gbufa, gbufb)
    sems = (sema, semb)

    def wait_gather(buf, sem):
        pltpu.make_async_copy(y_hbm.at[rblk.at[0, 0]], buf, sem).wait()

    def wait_rblk():
        pltpu.make_async_copy(rows_hbm.at[w * NBLK], rblk.at[0], semr).wait()

    pltpu.sync_copy(rows_hbm.at[w * NBLK], rblk.at[0])
    pltpu.async_copy(rows_hbm.at[w * NBLK + 1], rblk.at[1], semr)
    pltpu.async_copy(y_hbm.at[rblk.at[0, 0]], gbufa, sema)

    def block(b, cur, nxt):
        for t in range(NB):
            j = b * NB + t
            if t < NB - 1:
                pltpu.async_copy(y_hbm.at[rblk.at[cur, t + 1]],
                                 gbufs[(t + 1) % 2], sems[(t + 1) % 2])
            else:
                @pl.when(b < NBLK - 1)
                def _():
                    wait_rblk()
                    pltpu.async_copy(y_hbm.at[rblk.at[nxt, 0]],
                                     gbufs[(t + 1) % 2], sems[(t + 1) % 2])
            wait_gather(gbufs[t % 2], sems[t % 2])
            pltpu.sync_copy(gbufs[t % 2], z_s.at[cidx.at[j]], add=True)

        @pl.when(b + 2 < NBLK)
        def _():
            pltpu.async_copy(rows_hbm.at[w * NBLK + b + 2], rblk.at[cur], semr)

    @pl.loop(0, (NBLK - 1) // 2)
    def _(q):
        block(2 * q, 0, 1)
        block(2 * q + 1, 1, 0)

    block(NBLK - 1, 0, 1)

    plsc.subcore_barrier()
    pltpu.sync_copy(
        z_s.at[pl.ds(s * ROWS_PER_TILE, ROWS_PER_TILE)],
        out_hbm.at[pl.ds(c * NP + s * ROWS_PER_TILE, ROWS_PER_TILE)],
    )


def _prep_body(sdeg_ref, x_ref, dise_ref, y0_ref):
    deg = sdeg_ref[0:NP, 0:1] + sdeg_ref[NP:2 * NP, 0:1] + 1.0
    dis = lax.rsqrt(jnp.maximum(deg, 1.0))
    dise_ref[...] = dis
    y0_ref[...] = x_ref[...] * dis


_prep_call = pl.pallas_call(
    _prep_body,
    out_shape=(
        jax.ShapeDtypeStruct((NP, 1), jnp.float32),
        jax.ShapeDtypeStruct((NP, D), jnp.float32),
    ),
)


def _scale_body(s_ref, y_ref, dise_ref, t_ref, ynew_ref):
    dis = dise_ref[...]
    tmp = (s_ref[0:NP, :] + s_ref[NP:2 * NP, :] + y_ref[...]) * dis
    t_ref[...] = tmp
    ynew_ref[...] = tmp * dis


_scale_call = pl.pallas_call(
    _scale_body,
    out_shape=(
        jax.ShapeDtypeStruct((NP, D), jnp.float32),
        jax.ShapeDtypeStruct((NP, D), jnp.float32),
    ),
)


def _final_body(x_ref, t1_ref, t2_ref, t3_ref, t4_ref, w0t_ref, wts_ref,
                bsum_ref, fw_ref, wct_ref, bc_ref, out_ref):
    acc = jnp.dot(x_ref[...], w0t_ref[...], preferred_element_type=jnp.float32)
    for i, t_ref in enumerate((t1_ref, t2_ref, t3_ref, t4_ref)):
        acc = acc + jnp.dot(t_ref[...], wts_ref[i],
                            preferred_element_type=jnp.float32)
    acc = acc + bsum_ref[...]
    h = jnp.maximum(acc, 0.0)
    ew = jnp.exp(fw_ref[...])
    w0 = ew[0, 0] / (ew[0, 0] + ew[0, 1])
    out_ref[...] = jnp.dot(h * w0, wct_ref[...],
                           preferred_element_type=jnp.float32) + bc_ref[...]


_final_call = pl.pallas_call(
    _final_body,
    out_shape=jax.ShapeDtypeStruct((NP, 64), jnp.float32),
)


@jax.jit
def kernel(x, edge_index, W0, b0, W1, b1, W2, b2, W3, b3, W4, b4,
           fusion_weight, Wc, bc):
    pad = EP - E
    rows_p = jnp.concatenate([edge_index[0], jnp.zeros((pad,), jnp.int32)])
    pad_cols = N + (jnp.arange(pad, dtype=jnp.int32) % (NP - N))
    cols_p = jnp.concatenate([edge_index[1], pad_cols])
    rows3 = rows_p.reshape(32 * NBLK, NB, CH)
    cols3 = cols_p.reshape(32, K, CH)
    x_pad = jnp.pad(x, ((0, NP - N), (0, 0)))

    w0t = W0.T
    wts = jnp.stack([W1.T, W2.T, W3.T, W4.T])
    bsum = (b0 + b1 + b2 + b3 + b4).reshape(1, D)
    fw = fusion_weight.reshape(1, 2)
    wct = Wc.T
    bc2 = bc.reshape(1, 64)

    ones_feat = jnp.ones((NP, D), jnp.float32)
    sdeg = _spmv_kernel(ones_feat, rows3, cols3)
    dise, y = _prep_call(sdeg, x_pad)

    ts = []
    for _ in range(4):
        s_out = _spmv_kernel(y, rows3, cols3)
        t_i, y = _scale_call(s_out, y, dise)
        ts.append(t_i)

    logits = _final_call(x_pad, ts[0], ts[1], ts[2], ts[3],
                         w0t, wts, bsum, fw, wct, bc2)
    return logits[:N]


# speedup vs baseline: 13.4405x; 1.1601x over previous
import functools
import jax
import jax.numpy as jnp
from jax import lax
from jax.experimental import pallas as pl
from jax.experimental.pallas import tpu as pltpu
from jax.experimental.pallas import tpu_sc as plsc

N = 10000
NP = 10112
D = 128
CH = 112
NB = 6
NBLK0 = 20
NBLK1 = 10
NBLKS = 16 * (NBLK0 + NBLK1)
NBLKS_PAD = NBLKS + 16
E = 320000
EP = NBLKS * NB * CH
RPT = NP // 16

_mesh = plsc.VectorSubcoreMesh(core_axis_name="c", subcore_axis_name="s")


@functools.partial(
    pl.kernel,
    out_type=jax.ShapeDtypeStruct((2 * NP, D), jnp.float32),
    mesh=_mesh,
    scratch_types=[
        pltpu.VMEM((2, 2, NB, CH), jnp.int32),
        pltpu.VMEM((CH, D), jnp.float32),
        pltpu.VMEM((CH, D), jnp.float32),
        pltpu.VMEM_SHARED((NP, D), jnp.float32),
        pltpu.SemaphoreType.DMA,
        pltpu.SemaphoreType.DMA,
        pltpu.SemaphoreType.DMA,
    ],
)
def _spmv_kernel(y_hbm, rc_hbm, out_hbm, rcblk, gbufa, gbufb,
                 z_s, sema, semb, semr):
    c = lax.axis_index("c")
    s = lax.axis_index("s")
    nblk = jnp.where(c == 0, NBLK0, NBLK1)
    blkbase = jnp.where(c == 0, s * NBLK0, 16 * NBLK0 + s * NBLK1)

    zero16 = jnp.zeros((16,), jnp.float32)

    @pl.loop(0, CH)
    def _(i):
        for q in range(8):
            gbufa[i, pl.ds(16 * q, 16)] = zero16

    @pl.loop(0, 5)
    def _(k):
        pltpu.sync_copy(gbufa.at[pl.ds(0, 112)],
                        z_s.at[pl.ds(s * RPT + k * 112, 112)])

    pltpu.sync_copy(gbufa.at[pl.ds(0, 72)], z_s.at[pl.ds(s * RPT + 560, 72)])

    plsc.subcore_barrier()

    gbufs = (gbufa, gbufb)
    sems = (sema, semb)

    def wait_gather(buf, sem):
        pltpu.make_async_copy(y_hbm.at[rcblk.at[0, 0, 0]], buf, sem).wait()

    def wait_rcblk():
        pltpu.make_async_copy(rc_hbm.at[blkbase], rcblk.at[0], semr).wait()

    pltpu.sync_copy(rc_hbm.at[blkbase], rcblk.at[0])
    pltpu.async_copy(rc_hbm.at[blkbase + 1], rcblk.at[1], semr)
    pltpu.async_copy(y_hbm.at[rcblk.at[0, 0, 0]], gbufa, sema)

    def block(b, cur, nxt):
        for t in range(NB):
            if t < NB - 1:
                pltpu.async_copy(y_hbm.at[rcblk.at[cur, 0, t + 1]],
                                 gbufs[(t + 1) % 2], sems[(t + 1) % 2])
            else:
                @pl.when(b < nblk - 1)
                def _():
                    wait_rcblk()
                    pltpu.async_copy(y_hbm.at[rcblk.at[nxt, 0, 0]],
                                     gbufs[(t + 1) % 2], sems[(t + 1) % 2])
            wait_gather(gbufs[t % 2], sems[t % 2])
            pltpu.sync_copy(gbufs[t % 2], z_s.at[rcblk.at[cur, 1, t]],
                            add=True)

        @pl.when(b + 2 < nblk)
        def _():
            pltpu.async_copy(rc_hbm.at[blkbase + b + 2], rcblk.at[cur], semr)

    @pl.loop(0, nblk // 2)
    def _(q):
        block(2 * q, 0, 1)
        block(2 * q + 1, 1, 0)

    plsc.subcore_barrier()
    pltpu.sync_copy(
        z_s.at[pl.ds(s * RPT, RPT)],
        out_hbm.at[pl.ds(c * NP + s * RPT, RPT)],
    )


def _prep_body(sdeg_ref, x_ref, dise_ref, y0_ref):
    deg = sdeg_ref[0:NP, 0:1] + sdeg_ref[NP:2 * NP, 0:1] + 1.0
    dis = lax.rsqrt(jnp.maximum(deg, 1.0))
    dise_ref[...] = dis
    y0_ref[...] = x_ref[...] * dis


_prep_call = pl.pallas_call(
    _prep_body,
    out_shape=(
        jax.ShapeDtypeStruct((NP, 1), jnp.float32),
        jax.ShapeDtypeStruct((NP, D), jnp.float32),
    ),
)


def _scale_body(s_ref, y_ref, dise_ref, t_ref, ynew_ref):
    dis = dise_ref[...]
    tmp = (s_ref[0:NP, :] + s_ref[NP:2 * NP, :] + y_ref[...]) * dis
    t_ref[...] = tmp
    ynew_ref[...] = tmp * dis


_scale_call = pl.pallas_call(
    _scale_body,
    out_shape=(
        jax.ShapeDtypeStruct((NP, D), jnp.float32),
        jax.ShapeDtypeStruct((NP, D), jnp.float32),
    ),
)


def _final_body(x_ref, t1_ref, t2_ref, t3_ref, t4_ref, w0t_ref, wts_ref,
                bsum_ref, fw_ref, wct_ref, bc_ref, out_ref):
    acc = jnp.dot(x_ref[...], w0t_ref[...], preferred_element_type=jnp.float32)
    for i, t_ref in enumerate((t1_ref, t2_ref, t3_ref, t4_ref)):
        acc = acc + jnp.dot(t_ref[...], wts_ref[i],
                            preferred_element_type=jnp.float32)
    acc = acc + bsum_ref[...]
    h = jnp.maximum(acc, 0.0)
    ew = jnp.exp(fw_ref[...])
    w0 = ew[0, 0] / (ew[0, 0] + ew[0, 1])
    out_ref[...] = jnp.dot(h * w0, wct_ref[...],
                           preferred_element_type=jnp.float32) + bc_ref[...]


_final_call = pl.pallas_call(
    _final_body,
    out_shape=jax.ShapeDtypeStruct((NP, 64), jnp.float32),
)


@jax.jit
def kernel(x, edge_index, W0, b0, W1, b1, W2, b2, W3, b3, W4, b4,
           fusion_weight, Wc, bc):
    pad = EP - E
    rows_p = jnp.concatenate([edge_index[0], jnp.zeros((pad,), jnp.int32)])
    pad_cols = N + (jnp.arange(pad, dtype=jnp.int32) % (NP - N))
    cols_p = jnp.concatenate([edge_index[1], pad_cols])
    rows3 = rows_p.reshape(NBLKS, NB, CH)
    cols3 = cols_p.reshape(NBLKS, NB, CH)
    rc = jnp.stack([rows3, cols3], axis=1)
    x_pad = jnp.pad(x, ((0, NP - N), (0, 0)))

    w0t = W0.T
    wts = jnp.stack([W1.T, W2.T, W3.T, W4.T])
    bsum = (b0 + b1 + b2 + b3 + b4).reshape(1, D)
    fw = fusion_weight.reshape(1, 2)
    wct = Wc.T
    bc2 = bc.reshape(1, 64)

    ones_feat = jnp.ones((NP, D), jnp.float32)
    sdeg = _spmv_kernel(ones_feat, rc)
    dise, y = _prep_call(sdeg, x_pad)

    ts = []
    for _ in range(4):
        s_out = _spmv_kernel(y, rc)
        t_i, y = _scale_call(s_out, y, dise)
        ts.append(t_i)

    logits = _final_call(x_pad, ts[0], ts[1], ts[2], ts[3],
                        w0t, wts, bsum, fw, wct, bc2)
    return logits[:N]


# speedup vs baseline: 15.3280x; 1.1404x over previous
import functools
import jax
import jax.numpy as jnp
from jax import lax
from jax.experimental import pallas as pl
from jax.experimental.pallas import tpu as pltpu
from jax.experimental.pallas import tpu_sc as plsc

N = 10000
NP = 10112
D = 128
CH = 112
NB = 6
NBLK0 = 21
NBLK1 = 9
NBLKS = 16 * (NBLK0 + NBLK1)
NBLKS_PAD = NBLKS + 16
E = 320000
EP = NBLKS * NB * CH
RPT = NP // 16

_mesh = plsc.VectorSubcoreMesh(core_axis_name="c", subcore_axis_name="s")


@functools.partial(
    pl.kernel,
    out_type=jax.ShapeDtypeStruct((2 * NP, D), jnp.float32),
    mesh=_mesh,
    scratch_types=[
        pltpu.VMEM((2, 2, NB, CH), jnp.int32),
        pltpu.VMEM((CH, D), jnp.float32),
        pltpu.VMEM((CH, D), jnp.float32),
        pltpu.VMEM_SHARED((NP, D), jnp.float32),
        pltpu.SemaphoreType.DMA,
        pltpu.SemaphoreType.DMA,
        pltpu.SemaphoreType.DMA,
    ],
)
def _spmv_kernel(y_hbm, rc_hbm, out_hbm, rcblk, gbufa, gbufb,
                 z_s, sema, semb, semr):
    c = lax.axis_index("c")
    s = lax.axis_index("s")
    nblk = jnp.where(c == 0, NBLK0, NBLK1)
    blkbase = jnp.where(c == 0, s * NBLK0, 16 * NBLK0 + s * NBLK1)

    zero16 = jnp.zeros((16,), jnp.float32)

    @pl.loop(0, CH)
    def _(i):
        for q in range(8):
            gbufa[i, pl.ds(16 * q, 16)] = zero16

    @pl.loop(0, 5)
    def _(k):
        pltpu.sync_copy(gbufa.at[pl.ds(0, 112)],
                        z_s.at[pl.ds(s * RPT + k * 112, 112)])

    pltpu.sync_copy(gbufa.at[pl.ds(0, 72)], z_s.at[pl.ds(s * RPT + 560, 72)])

    plsc.subcore_barrier()

    gbufs = (gbufa, gbufb)
    sems = (sema, semb)

    def wait_gather(buf, sem):
        pltpu.make_async_copy(y_hbm.at[rcblk.at[0, 0, 0]], buf, sem).wait()

    def wait_rcblk():
        pltpu.make_async_copy(rc_hbm.at[blkbase], rcblk.at[0], semr).wait()

    pltpu.sync_copy(rc_hbm.at[blkbase], rcblk.at[0])
    pltpu.async_copy(rc_hbm.at[blkbase + 1], rcblk.at[1], semr)
    pltpu.async_copy(y_hbm.at[rcblk.at[0, 0, 0]], gbufa, sema)

    def block(b, cur, nxt):
        for t in range(NB):
            if t < NB - 1:
                pltpu.async_copy(y_hbm.at[rcblk.at[cur, 0, t + 1]],
                                 gbufs[(t + 1) % 2], sems[(t + 1) % 2])
            else:
                @pl.when(b < nblk - 1)
                def _():
                    wait_rcblk()
                    pltpu.async_copy(y_hbm.at[rcblk.at[nxt, 0, 0]],
                                     gbufs[(t + 1) % 2], sems[(t + 1) % 2])
            wait_gather(gbufs[t % 2], sems[t % 2])
            pltpu.sync_copy(gbufs[t % 2], z_s.at[rcblk.at[cur, 1, t]],
                            add=True)

        @pl.when(b + 2 < nblk)
        def _():
            pltpu.async_copy(rc_hbm.at[blkbase + b + 2], rcblk.at[cur], semr)

    @pl.loop(0, nblk // 2)
    def _(q):
        block(2 * q, 0, 1)
        block(2 * q + 1, 1, 0)

    @pl.when(nblk % 2 == 1)
    def _():
        block(nblk - 1, 0, 1)

    plsc.subcore_barrier()
    pltpu.sync_copy(
        z_s.at[pl.ds(s * RPT, RPT)],
        out_hbm.at[pl.ds(c * NP + s * RPT, RPT)],
    )


@functools.partial(
    pl.kernel,
    out_type=jax.ShapeDtypeStruct((2 * NP, D), jnp.float32),
    mesh=_mesh,
    scratch_types=[
        pltpu.VMEM((2, 2, NB, CH), jnp.int32),
        pltpu.VMEM((CH, D), jnp.float32),
        pltpu.VMEM_SHARED((NP, D), jnp.float32),
        pltpu.SemaphoreType.DMA,
    ],
)
def _deg_kernel(rc_hbm, out_hbm, rcblk, gones, z_s, semr):
    c = lax.axis_index("c")
    s = lax.axis_index("s")
    nblk = jnp.where(c == 0, NBLK0, NBLK1)
    blkbase = jnp.where(c == 0, s * NBLK0, 16 * NBLK0 + s * NBLK1)

    zero16 = jnp.zeros((16,), jnp.float32)
    ones16 = jnp.ones((16,), jnp.float32)

    @pl.loop(0, CH)
    def _(i):
        for q in range(8):
            gones[i, pl.ds(16 * q, 16)] = zero16

    @pl.loop(0, 5)
    def _(k):
        pltpu.sync_copy(gones.at[pl.ds(0, 112)],
                        z_s.at[pl.ds(s * RPT + k * 112, 112)])

    pltpu.sync_copy(gones.at[pl.ds(0, 72)], z_s.at[pl.ds(s * RPT + 560, 72)])

    @pl.loop(0, CH)
    def _(i):
        gones[i, pl.ds(0, 16)] = ones16

    plsc.subcore_barrier()

    def wait_rcblk():
        pltpu.make_async_copy(rc_hbm.at[blkbase], rcblk.at[0], semr).wait()

    pltpu.sync_copy(rc_hbm.at[blkbase], rcblk.at[0])
    pltpu.async_copy(rc_hbm.at[blkbase + 1], rcblk.at[1], semr)

    def block(b, cur, nxt):
        for t in range(NB):
            pltpu.sync_copy(gones, z_s.at[rcblk.at[cur, 1, t]], add=True)

        @pl.when(b < nblk - 2)
        def _():
            wait_rcblk()
            pltpu.async_copy(rc_hbm.at[blkbase + b + 2], rcblk.at[cur], semr)

        @pl.when(b == nblk - 2)
        def _():
            wait_rcblk()

    @pl.loop(0, nblk // 2)
    def _(q):
        block(2 * q, 0, 1)
        block(2 * q + 1, 1, 0)

    @pl.when(nblk % 2 == 1)
    def _():
        block(nblk - 1, 0, 1)

    plsc.subcore_barrier()
    pltpu.sync_copy(
        z_s.at[pl.ds(s * RPT, RPT)],
        out_hbm.at[pl.ds(c * NP + s * RPT, RPT)],
    )


def _prep_body(sdeg_ref, x_ref, dise_ref, y0_ref):
    deg = sdeg_ref[0:NP, 0:1] + sdeg_ref[NP:2 * NP, 0:1] + 1.0
    dis = lax.rsqrt(jnp.maximum(deg, 1.0))
    dise_ref[...] = dis
    y0_ref[...] = x_ref[...] * dis


_prep_call = pl.pallas_call(
    _prep_body,
    out_shape=(
        jax.ShapeDtypeStruct((NP, 1), jnp.float32),
        jax.ShapeDtypeStruct((NP, D), jnp.float32),
    ),
)


def _scale_body(s_ref, y_ref, dise_ref, t_ref, ynew_ref):
    dis = dise_ref[...]
    tmp = (s_ref[0:NP, :] + s_ref[NP:2 * NP, :] + y_ref[...]) * dis
    t_ref[...] = tmp
    ynew_ref[...] = tmp * dis


_scale_call = pl.pallas_call(
    _scale_body,
    out_shape=(
        jax.ShapeDtypeStruct((NP, D), jnp.float32),
        jax.ShapeDtypeStruct((NP, D), jnp.float32),
    ),
)


def _final_body(x_ref, t1_ref, t2_ref, t3_ref, t4_ref, w0t_ref, wts_ref,
                bsum_ref, fw_ref, wct_ref, bc_ref, out_ref):
    acc = jnp.dot(x_ref[...], w0t_ref[...], preferred_element_type=jnp.float32)
    for i, t_ref in enumerate((t1_ref, t2_ref, t3_ref, t4_ref)):
        acc = acc + jnp.dot(t_ref[...], wts_ref[i],
                            preferred_element_type=jnp.float32)
    acc = acc + bsum_ref[...]
    h = jnp.maximum(acc, 0.0)
    ew = jnp.exp(fw_ref[...])
    w0 = ew[0, 0] / (ew[0, 0] + ew[0, 1])
    out_ref[...] = jnp.dot(h * w0, wct_ref[...],
                           preferred_element_type=jnp.float32) + bc_ref[...]


_final_call = pl.pallas_call(
    _final_body,
    out_shape=jax.ShapeDtypeStruct((NP, 64), jnp.float32),
)


@jax.jit
def kernel(x, edge_index, W0, b0, W1, b1, W2, b2, W3, b3, W4, b4,
           fusion_weight, Wc, bc):
    pad = EP - E
    rows_p = jnp.concatenate([edge_index[0], jnp.zeros((pad,), jnp.int32)])
    pad_cols = N + (jnp.arange(pad, dtype=jnp.int32) % (NP - N))
    cols_p = jnp.concatenate([edge_index[1], pad_cols])
    rows3 = rows_p.reshape(NBLKS, NB, CH)
    cols3 = cols_p.reshape(NBLKS, NB, CH)
    rc = jnp.stack([rows3, cols3], axis=1)
    x_pad = jnp.pad(x, ((0, NP - N), (0, 0)))

    w0t = W0.T
    wts = jnp.stack([W1.T, W2.T, W3.T, W4.T])
    bsum = (b0 + b1 + b2 + b3 + b4).reshape(1, D)
    fw = fusion_weight.reshape(1, 2)
    wct = Wc.T
    bc2 = bc.reshape(1, 64)

    sdeg = _deg_kernel(rc)
    dise, y = _prep_call(sdeg, x_pad)

    ts = []
    for _ in range(4):
        s_out = _spmv_kernel(y, rc)
        t_i, y = _scale_call(s_out, y, dise)
        ts.append(t_i)

    logits = _final_call(x_pad, ts[0], ts[1], ts[2], ts[3],
                        w0t, wts, bsum, fw, wct, bc2)
    return logits[:N]
